# baseline (speedup 1.0000x reference)
import functools

import jax
import jax.numpy as jnp
from jax import lax
from jax.experimental import pallas as pl
from jax.experimental.pallas import tpu as pltpu
from jax.experimental.pallas import tpu_sc as plsc


def kernel(word, pos, word_table, pos_table):
    B, S = word.shape
    V, D = word_table.shape
    DP = pos_table.shape[1]
    G = 128
    NC, NS = 2, 16
    NW = NC * NS
    PACK = 128 // D

    word_t = word.T
    pos_t = pos.T
    ptab_t = jnp.pad(pos_table.T, ((0, 0), (0, 128 - pos_table.shape[0])))
    wtab4 = word_table.reshape(V // PACK, 128)

    mesh = plsc.VectorSubcoreMesh(core_axis_name="c", subcore_axis_name="s")

    @functools.partial(
        pl.kernel,
        mesh=mesh,
        compiler_params=pltpu.CompilerParams(
            use_tc_tiling_on_sc=True, needs_layout_passes=False),
        out_type=jax.ShapeDtypeStruct((S, DP + D, B), jnp.float32),
        scratch_types=[
            pltpu.VMEM((S, G), jnp.int32),
            pltpu.VMEM((S, G), jnp.int32),
            pltpu.VMEM((32, 128), jnp.float32),
            pltpu.VMEM((4, G), jnp.int32),
            pltpu.VMEM((4, G), jnp.int32),
            pltpu.VMEM((4, G, 128), jnp.float32),
            pltpu.VMEM((2, DP + D, G), jnp.float32),
            pltpu.SemaphoreType.DMA,
            pltpu.SemaphoreType.DMA,
            pltpu.SemaphoreType.DMA,
            pltpu.SemaphoreType.DMA,
            pltpu.SemaphoreType.DMA,
            pltpu.SemaphoreType.DMA,
        ],
    )
    def emb_kernel(widx_hbm, pidx_hbm, ptab_hbm, wtab_hbm, out_hbm,
                   widx_v, pidx_v, ptab_v, gidx_v, qoff_v, wrows_v, obuf_v,
                   g0, g1, g2, g3, w0, w1):
        wid = lax.axis_index("s") * NC + lax.axis_index("c")
        lane0 = wid * G

        pltpu.sync_copy(widx_hbm.at[:, pl.ds(lane0, G)], widx_v)
        pltpu.sync_copy(pidx_hbm.at[:, pl.ds(lane0, G)], pidx_v)
        pltpu.sync_copy(ptab_hbm, ptab_v)

        gsems = (g0, g1, g2, g3)
        wsems = (w0, w1)

        def prep_gather(j, sl):
            for m in range(G // 16):
                v = widx_v[j, pl.ds(16 * m, 16)]
                gidx_v[sl, pl.ds(16 * m, 16)] = lax.shift_right_logical(v, 2)
                qoff_v[sl, pl.ds(16 * m, 16)] = lax.shift_left(
                    lax.bitwise_and(v, 3), 5)

        def start_gather(sl):
            pltpu.async_copy(wtab_hbm.at[gidx_v.at[sl]], wrows_v.at[sl],
                             gsems[sl])

        def wait_gather(sl):
            pltpu.make_async_copy(wtab_hbm.at[gidx_v.at[sl]],
                                  wrows_v.at[sl], gsems[sl]).wait()

        def extract(j, sl, osl):
            def fbody(f, carry):
                fvec = jnp.zeros((16,), jnp.int32) + f
                for m in range(G // 16):
                    pvec = pidx_v[j, pl.ds(16 * m, 16)]
                    qvec = qoff_v[sl, pl.ds(16 * m, 16)]
                    kvec = lax.iota(jnp.int32, 16) + 16 * m
                    pv = plsc.load_gather(ptab_v, [fvec, pvec])
                    obuf_v[osl, f, pl.ds(16 * m, 16)] = pv
                    wv = plsc.load_gather(wrows_v.at[sl], [kvec, qvec + f])
                    obuf_v[osl, DP + f, pl.ds(16 * m, 16)] = wv
                return carry

            lax.fori_loop(0, DP, fbody, 0)

        def start_write(j, sl):
            pltpu.async_copy(obuf_v.at[sl],
                             out_hbm.at[j, :, pl.ds(lane0, G)], wsems[sl])

        def wait_write(sl):
            pltpu.make_async_copy(obuf_v.at[sl],
                                  out_hbm.at[0, :, pl.ds(lane0, G)],
                                  wsems[sl]).wait()

        for j0 in range(3):
            prep_gather(j0, j0)
            start_gather(j0)

        def substep(j, sl, osl):
            wait_gather(sl)

            @pl.when(j + 3 < S)
            def _():
                prep_gather(j + 3, (sl + 3) % 4)
                start_gather((sl + 3) % 4)

            @pl.when(j >= 2)
            def _():
                wait_write(osl)
            extract(j, sl, osl)
            start_write(j, osl)

        def step(i, carry):
            for sl in (0, 1, 2, 3):
                substep(4 * i + sl, sl, sl % 2)
            return carry

        lax.fori_loop(0, S // 4, step, 0)
        for j in range(4 * (S // 4), S):
            substep(j, j % 4, j % 2)
        wait_write(0)
        wait_write(1)

    out = emb_kernel(word_t, pos_t, ptab_t, wtab4)
    return out.transpose(2, 0, 1)


# speedup vs baseline: 1.0808x; 1.0808x over previous
import functools

import jax
import jax.numpy as jnp
from jax import lax
from jax.experimental import pallas as pl
from jax.experimental.pallas import tpu as pltpu
from jax.experimental.pallas import tpu_sc as plsc


def kernel(word, pos, word_table, pos_table):
    B, S = word.shape
    V, D = word_table.shape
    DP = pos_table.shape[1]
    G = 128
    NC, NS = 2, 16
    NW = NC * NS
    PACK = 128 // D

    word_t = word.T
    pos_t = pos.T
    ptab_t = jnp.pad(pos_table.T, ((0, 0), (0, 128 - pos_table.shape[0])))
    wtab4 = word_table.reshape(V // PACK, 128)

    mesh = plsc.VectorSubcoreMesh(core_axis_name="c", subcore_axis_name="s")

    @functools.partial(
        pl.kernel,
        mesh=mesh,
        compiler_params=pltpu.CompilerParams(
            use_tc_tiling_on_sc=True, needs_layout_passes=False),
        out_type=jax.ShapeDtypeStruct((S, DP + D, B), jnp.float32),
        scratch_types=[
            pltpu.VMEM((S, G), jnp.int32),
            pltpu.VMEM((S, G), jnp.int32),
            pltpu.VMEM((32, 128), jnp.float32),
            pltpu.VMEM((2, G), jnp.int32),
            pltpu.VMEM((2, G), jnp.int32),
            pltpu.VMEM((2, G, 128), jnp.float32),
            pltpu.VMEM((2, DP + D, G), jnp.float32),
            pltpu.SemaphoreType.DMA,
            pltpu.SemaphoreType.DMA,
            pltpu.SemaphoreType.DMA,
            pltpu.SemaphoreType.DMA,
        ],
    )
    def emb_kernel(widx_hbm, pidx_hbm, ptab_hbm, wtab_hbm, out_hbm,
                   widx_v, pidx_v, ptab_v, gidx_v, qoff_v, wrows_v, obuf_v,
                   g0, g1, w0, w1):
        wid = lax.axis_index("s") * NC + lax.axis_index("c")
        lane0 = wid * G

        pltpu.sync_copy(widx_hbm.at[:, pl.ds(lane0, G)], widx_v)
        pltpu.sync_copy(pidx_hbm.at[:, pl.ds(lane0, G)], pidx_v)
        pltpu.sync_copy(ptab_hbm, ptab_v)

        gsems = (g0, g1)
        wsems = (w0, w1)

        def prep_gather(j, sl):
            for m in range(G // 16):
                v = widx_v[j, pl.ds(16 * m, 16)]
                gidx_v[sl, pl.ds(16 * m, 16)] = lax.shift_right_logical(v, 2)
                qoff_v[sl, pl.ds(16 * m, 16)] = lax.shift_left(
                    lax.bitwise_and(v, 3), 5)

        def start_gather(sl):
            pltpu.async_copy(wtab_hbm.at[gidx_v.at[sl]], wrows_v.at[sl],
                             gsems[sl])

        def wait_gather(sl):
            pltpu.make_async_copy(wtab_hbm.at[gidx_v.at[sl]],
                                  wrows_v.at[sl], gsems[sl]).wait()

        def extract_pos(j, sl):
            for m in range(G // 16):
                pvec = pidx_v[j, pl.ds(16 * m, 16)]
                for f in range(DP):
                    fvec = jnp.full((16,), f, jnp.int32)
                    pv = plsc.load_gather(ptab_v, [fvec, pvec])
                    obuf_v[sl, f, pl.ds(16 * m, 16)] = pv

        def extract_word(j, sl):
            for m in range(G // 16):
                qvec = qoff_v[sl, pl.ds(16 * m, 16)]
                kvec = lax.iota(jnp.int32, 16) + 16 * m
                for f in range(D):
                    wv = plsc.load_gather(wrows_v.at[sl], [kvec, qvec + f])
                    obuf_v[sl, DP + f, pl.ds(16 * m, 16)] = wv

        def start_write(j, sl):
            pltpu.async_copy(obuf_v.at[sl],
                             out_hbm.at[j, :, pl.ds(lane0, G)], wsems[sl])

        def wait_write(sl):
            pltpu.make_async_copy(obuf_v.at[sl],
                                  out_hbm.at[0, :, pl.ds(lane0, G)],
                                  wsems[sl]).wait()

        prep_gather(0, 0)
        start_gather(0)

        def step(i, carry):
            for sl in (0, 1):
                j = 2 * i + sl

                @pl.when(j >= 2)
                def _():
                    wait_write(sl)
                extract_pos(j, sl)
                wait_gather(sl)

                @pl.when(j + 1 < S)
                def _():
                    prep_gather(j + 1, 1 - sl)
                    start_gather(1 - sl)
                extract_word(j, sl)
                start_write(j, sl)
            return carry

        lax.fori_loop(0, S // 2, step, 0)
        wait_write(0)
        wait_write(1)

    out = emb_kernel(word_t, pos_t, ptab_t, wtab4)
    return out.transpose(2, 0, 1)


# speedup vs baseline: 1.1532x; 1.0669x over previous
import functools

import jax
import jax.numpy as jnp
from jax import lax
from jax.experimental import pallas as pl
from jax.experimental.pallas import tpu as pltpu
from jax.experimental.pallas import tpu_sc as plsc


def kernel(word, pos, word_table, pos_table):
    B, S = word.shape
    V, D = word_table.shape
    DP = pos_table.shape[1]
    G = 128
    NC, NS = 2, 16
    NW = NC * NS
    PACK = 128 // D

    word_t = word.T
    pos_t = pos.T
    ptab_t = jnp.pad(pos_table.T, ((0, 0), (0, 128 - pos_table.shape[0])))
    wtab4 = word_table.reshape(V // PACK, 128)

    mesh = plsc.VectorSubcoreMesh(core_axis_name="c", subcore_axis_name="s")

    @functools.partial(
        pl.kernel,
        mesh=mesh,
        compiler_params=pltpu.CompilerParams(
            use_tc_tiling_on_sc=True, needs_layout_passes=False),
        out_type=jax.ShapeDtypeStruct((S, DP + D, B), jnp.float32),
        scratch_types=[
            pltpu.VMEM((S, G), jnp.int32),
            pltpu.VMEM((S, G), jnp.int32),
            pltpu.VMEM((32, 128), jnp.float32),
            pltpu.VMEM((2, G), jnp.int32),
            pltpu.VMEM((2, G), jnp.int32),
            pltpu.VMEM((2, G, 128), jnp.float32),
            pltpu.VMEM((2, DP + D, G), jnp.float32),
            pltpu.SemaphoreType.DMA,
            pltpu.SemaphoreType.DMA,
            pltpu.SemaphoreType.DMA,
            pltpu.SemaphoreType.DMA,
        ],
    )
    def emb_kernel(widx_hbm, pidx_hbm, ptab_hbm, wtab_hbm, out_hbm,
                   widx_v, pidx_v, ptab_v, gidx_v, qoff_v, wrows_v, obuf_v,
                   g0, g1, w0, w1):
        wid = lax.axis_index("s") * NC + lax.axis_index("c")
        lane0 = wid * G

        pltpu.sync_copy(widx_hbm.at[:, pl.ds(lane0, G)], widx_v)
        pltpu.sync_copy(pidx_hbm.at[:, pl.ds(lane0, G)], pidx_v)
        pltpu.sync_copy(ptab_hbm, ptab_v)

        gsems = (g0, g1)
        wsems = (w0, w1)

        def prep_gather(j, sl):
            for m in range(G // 16):
                v = widx_v[j, pl.ds(16 * m, 16)]
                gidx_v[sl, pl.ds(16 * m, 16)] = lax.shift_right_logical(v, 2)
                qoff_v[sl, pl.ds(16 * m, 16)] = lax.shift_left(
                    lax.bitwise_and(v, 3), 5)

        def start_gather(sl):
            pltpu.async_copy(wtab_hbm.at[gidx_v.at[sl]], wrows_v.at[sl],
                             gsems[sl])

        def wait_gather(sl):
            pltpu.make_async_copy(wtab_hbm.at[gidx_v.at[sl]],
                                  wrows_v.at[sl], gsems[sl]).wait()

        def extract_all(j, sl):
            for m in range(G // 16):
                pvec = pidx_v[j, pl.ds(16 * m, 16)]
                qvec = qoff_v[sl, pl.ds(16 * m, 16)]
                kvec = lax.iota(jnp.int32, 16) + 16 * m
                for f in range(DP):
                    fvec = jnp.full((16,), f, jnp.int32)
                    pv = plsc.load_gather(ptab_v, [fvec, pvec])
                    wv = plsc.load_gather(wrows_v.at[sl], [kvec, qvec + f])
                    obuf_v[sl, f, pl.ds(16 * m, 16)] = pv
                    obuf_v[sl, DP + f, pl.ds(16 * m, 16)] = wv

        def start_write(j, sl):
            pltpu.async_copy(obuf_v.at[sl],
                             out_hbm.at[j, :, pl.ds(lane0, G)], wsems[sl])

        def wait_write(sl):
            pltpu.make_async_copy(obuf_v.at[sl],
                                  out_hbm.at[0, :, pl.ds(lane0, G)],
                                  wsems[sl]).wait()

        prep_gather(0, 0)
        start_gather(0)

        def step(i, carry):
            for sl in (0, 1):
                j = 2 * i + sl

                wait_gather(sl)

                @pl.when(j + 1 < S)
                def _():
                    prep_gather(j + 1, 1 - sl)
                    start_gather(1 - sl)

                @pl.when(j >= 2)
                def _():
                    wait_write(sl)
                extract_all(j, sl)
                start_write(j, sl)
            return carry

        lax.fori_loop(0, S // 2, step, 0)
        wait_write(0)
        wait_write(1)

    out = emb_kernel(word_t, pos_t, ptab_t, wtab4)
    return out.transpose(2, 0, 1)
